# baseline (speedup 1.0000x reference)
import functools

import jax
import jax.numpy as jnp
from jax import lax
from jax.experimental import pallas as pl
from jax.experimental.pallas import tpu as pltpu
from jax.experimental.pallas import tpu_sc as plsc

_L = 16
_NC = 2
_NS = 16
_NW = _NC * _NS


@functools.partial(jax.jit, static_argnums=(1,))
def _run(flat, S):
    R = flat.shape[0] // (2 * S + 3)
    NB = S + 1
    NO = 2 * NB
    RW = R // _NW
    SB = 128
    NSB = RW // SB
    GPB = SB // _L
    NBL = NB * _L

    mesh = plsc.VectorSubcoreMesh(core_axis_name="c", subcore_axis_name="s")

    OFF_S = R * S
    OFF_E = 2 * R * S
    OFF_N = OFF_E + R
    OFF_F = OFF_N + R

    @functools.partial(
        pl.kernel,
        out_type=jax.ShapeDtypeStruct((R * NO,), jnp.float32),
        mesh=mesh,
        compiler_params=pltpu.CompilerParams(needs_layout_passes=False),
        scratch_types=[
            pltpu.VMEM((SB * S,), jnp.float32),
            pltpu.VMEM((SB * S,), jnp.float32),
            pltpu.VMEM((SB * S,), jnp.float32),
            pltpu.VMEM((SB * S,), jnp.float32),
            pltpu.VMEM((SB,), jnp.float32),
            pltpu.VMEM((SB,), jnp.float32),
            pltpu.VMEM((SB,), jnp.float32),
            pltpu.VMEM((SB,), jnp.float32),
            pltpu.VMEM((SB,), jnp.float32),
            pltpu.VMEM((SB,), jnp.float32),
            pltpu.VMEM((SB * NO,), jnp.float32),
            pltpu.VMEM((SB * NO,), jnp.float32),
            pltpu.SemaphoreType.DMA,
            pltpu.SemaphoreType.DMA,
            pltpu.SemaphoreType.DMA,
            pltpu.SemaphoreType.DMA,
            pltpu.VMEM((NBL,), jnp.float32),
            pltpu.VMEM((NBL,), jnp.float32),
            pltpu.VMEM((NBL,), jnp.float32),
            pltpu.VMEM((NBL,), jnp.float32),
            pltpu.VMEM(((NB + 1) * _L,), jnp.int32),
            pltpu.VMEM(((NB + 1) * _L,), jnp.int32),
        ],
    )
    def body(x_hbm, out_hbm,
             wblk0, wblk1, sblk0, sblk1, eblk0, eblk1, nblk0, nblk1,
             fblk0, fblk1, outblk0, outblk1, isem0, isem1, osem0, osem1,
             cdf0, cdf1, eb0, eb1, h0, h1):
        cdfs, ebinss, hists = (cdf0, cdf1), (eb0, eb1), (h0, h1)
        bufs = ((wblk0, sblk0, eblk0, nblk0, fblk0), (wblk1, sblk1, eblk1, nblk1, fblk1))
        outblks = (outblk0, outblk1)
        isems, osems = (isem0, isem1), (osem0, osem1)
        wid = lax.axis_index("s") * _NC + lax.axis_index("c")
        base = wid * RW
        lane = lax.iota(jnp.int32, _L)
        ones_i = jnp.ones((_L,), jnp.int32)
        zeros_i = jnp.zeros((_L,), jnp.int32)
        zeros_f = jnp.zeros((_L,), jnp.float32)
        def make_gpair(wblk, sblk, eblk, nblk, fblk, outblk):
          def gpair(gp, _):
            st = []
            for t in (0, 1):
                g16 = (gp * 2 + t) * _L
                rows = g16 + lane
                near = nblk[pl.ds(g16, _L)]
                far = fblk[pl.ds(g16, _L)]
                st.append(dict(
                    g16=g16, rows=rows, rowsS=rows * S, rowsO=rows * NO,
                    near=near, fmn=far - near,
                    cdf=cdfs[t], ebins=ebinss[t], hist=hists[t],
                ))

            def pa(s, accs):
                out = []
                for t in (0, 1):
                    d = st[t]
                    ww = plsc.load_gather(wblk, [d["rowsS"] + s])
                    ee = plsc.load_gather(sblk, [d["rowsS"] + s])
                    acc = accs[t] + (ww + 0.01)
                    d["cdf"][pl.ds((s + 1) * _L, _L)] = acc
                    d["ebins"][pl.ds(s * _L, _L)] = ee
                    d["hist"][pl.ds(s * _L, _L)] = zeros_i
                    out.append(acc)
                return tuple(out)

            wss = plsc.parallel_loop(0, S, 1, unroll=4, carry=(zeros_f, zeros_f))(pa)
            pads = []
            for t in (0, 1):
                d = st[t]
                d["hist"][pl.ds(S * _L, _L)] = zeros_i
                d["hist"][pl.ds((S + 1) * _L, _L)] = zeros_i
                d["cdf"][pl.ds(0, _L)] = zeros_f
                d["ebins"][pl.ds(S * _L, _L)] = eblk[pl.ds(d["g16"], _L)]
                ws = wss[t]
                pad = jnp.maximum(0.0, 1e-5 - ws)
                pads.append((pad * (1.0 / S), 1.0 / (ws + pad)))

            def pb(k, carry):
                kf = k.astype(jnp.float32)
                for t in (0, 1):
                    d = st[t]
                    pad64, inv = pads[t]
                    cum = d["cdf"][pl.ds(k * _L, _L)]
                    c = jnp.minimum(1.0, (cum + kf * pad64) * inv)
                    d["cdf"][pl.ds(k * _L, _L)] = c
                    y = c * NB - 0.5
                    tr = y.astype(jnp.int32)
                    m = jnp.where(y > tr.astype(jnp.float32), tr + 1, tr)
                    plsc.addupdate_scatter(d["hist"], [m * _L + lane], ones_i)
                    ek = d["ebins"][pl.ds(k * _L, _L)]
                    val = d["near"] + ek * d["fmn"]
                    plsc.store_scatter(outblk, [d["rowsO"] + (m + k)], val)
                return carry

            plsc.parallel_loop(0, NB, 1, unroll=4, carry=jnp.int32(0))(pb)

            def pc(j, accs):
                u = (j.astype(jnp.float32) * 2.0 + 1.0) * (1.0 / NO)
                jp1 = j + 1
                out = []
                for t in (0, 1):
                    d = st[t]
                    acc = accs[t] + d["hist"][pl.ds(j * _L, _L)]
                    below = jnp.minimum(acc - 1, S)
                    above = jnp.minimum(acc, S)
                    bidx = below * _L + lane
                    aidx = above * _L + lane
                    c0 = plsc.load_gather(d["cdf"], [bidx])
                    c1 = plsc.load_gather(d["cdf"], [aidx])
                    e0 = plsc.load_gather(d["ebins"], [bidx])
                    e1 = plsc.load_gather(d["ebins"], [aidx])
                    num = u - c0
                    den = c1 - c0
                    tt = jnp.clip(num / den, 0.0, 1.0)
                    tt = jnp.where(den == 0.0, jnp.where(num > 0.0, 1.0, 0.0), tt)
                    bval = e0 + tt * (e1 - e0)
                    val = d["near"] + bval * d["fmn"]
                    plsc.store_scatter(outblk, [d["rowsO"] + (below + jp1)], val)
                    out.append(acc)
                return tuple(out)

            plsc.parallel_loop(0, NB, 1, unroll=4, carry=(zeros_i, zeros_i))(pc)
            return 0
          return gpair

        gpairs = tuple(make_gpair(*bufs[p], outblks[p]) for p in (0, 1))

        def start_in(p, sb):
            wblk, sblk, eblk, nblk, fblk = bufs[p]

            @pl.when(sb < NSB)
            def _():
                row0 = base + sb * SB
                pltpu.async_copy(x_hbm.at[pl.ds(row0 * S, SB * S)], wblk, isems[p])
                pltpu.async_copy(
                    x_hbm.at[pl.ds(OFF_S + row0 * S, SB * S)], sblk, isems[p])
                pltpu.async_copy(x_hbm.at[pl.ds(OFF_E + row0, SB)], eblk, isems[p])
                pltpu.async_copy(x_hbm.at[pl.ds(OFF_N + row0, SB)], nblk, isems[p])
                pltpu.async_copy(x_hbm.at[pl.ds(OFF_F + row0, SB)], fblk, isems[p])

        def wait_in(p):
            wblk, sblk, eblk, nblk, fblk = bufs[p]
            pltpu.make_async_copy(x_hbm.at[pl.ds(0, SB * S)], wblk, isems[p]).wait()
            pltpu.make_async_copy(x_hbm.at[pl.ds(0, SB * S)], sblk, isems[p]).wait()
            pltpu.make_async_copy(x_hbm.at[pl.ds(0, SB)], eblk, isems[p]).wait()
            pltpu.make_async_copy(x_hbm.at[pl.ds(0, SB)], nblk, isems[p]).wait()
            pltpu.make_async_copy(x_hbm.at[pl.ds(0, SB)], fblk, isems[p]).wait()

        def wait_out(p):
            pltpu.make_async_copy(
                x_hbm.at[pl.ds(0, SB * NO)], outblks[p], osems[p]).wait()

        start_in(0, base * 0)

        def halfstep(h, _):
            for p in (0, 1):
                sb = h * 2 + p
                start_in(1 - p, sb + 1)
                wait_in(p)

                @pl.when(h > 0)
                def _():
                    wait_out(p)

                lax.fori_loop(0, GPB // 2, gpairs[p], 0)
                row0 = base + sb * SB
                pltpu.async_copy(
                    outblks[p], out_hbm.at[pl.ds(row0 * NO, SB * NO)], osems[p])
            return 0

        lax.fori_loop(0, NSB // 2, halfstep, 0)
        wait_out(0)
        wait_out(1)

    return body(flat).reshape(R, NO)


def kernel(weights, spacing_starts, spacing_ends, nears, fars, num_samples=64):
    R, S = weights.shape[0], weights.shape[1]
    flat = jnp.concatenate([
        weights.reshape(R * S),
        spacing_starts.reshape(R * S),
        spacing_ends[:, -1, 0],
        nears.reshape(R),
        fars.reshape(R),
    ])
    return _run(flat, S)


# speedup vs baseline: 1.0621x; 1.0621x over previous
import functools

import jax
import jax.numpy as jnp
from jax import lax
from jax.experimental import pallas as pl
from jax.experimental.pallas import tpu as pltpu
from jax.experimental.pallas import tpu_sc as plsc

_L = 16
_NC = 2
_NS = 16
_NW = _NC * _NS


@functools.partial(jax.jit, static_argnums=(5,))
def _run(w2, s2, elast, nvec, fvec, S):
    R = w2.shape[0] // S
    NB = S + 1
    NO = 2 * NB
    RW = R // _NW
    SB = 128
    NSB = RW // SB
    GPB = SB // _L
    NBL = NB * _L

    mesh = plsc.VectorSubcoreMesh(core_axis_name="c", subcore_axis_name="s")

    @functools.partial(
        pl.kernel,
        out_type=jax.ShapeDtypeStruct((R * NO,), jnp.float32),
        mesh=mesh,
        compiler_params=pltpu.CompilerParams(needs_layout_passes=False),
        scratch_types=[
            pltpu.VMEM((SB * S,), jnp.float32),
            pltpu.VMEM((SB * S,), jnp.float32),
            pltpu.VMEM((SB * S,), jnp.float32),
            pltpu.VMEM((SB * S,), jnp.float32),
            pltpu.VMEM((SB,), jnp.float32),
            pltpu.VMEM((SB,), jnp.float32),
            pltpu.VMEM((SB,), jnp.float32),
            pltpu.VMEM((SB,), jnp.float32),
            pltpu.VMEM((SB,), jnp.float32),
            pltpu.VMEM((SB,), jnp.float32),
            pltpu.VMEM((SB * NO,), jnp.float32),
            pltpu.VMEM((SB * NO,), jnp.float32),
            pltpu.SemaphoreType.DMA,
            pltpu.SemaphoreType.DMA,
            pltpu.SemaphoreType.DMA,
            pltpu.SemaphoreType.DMA,
            pltpu.VMEM((NBL,), jnp.float32),
            pltpu.VMEM((NBL,), jnp.float32),
            pltpu.VMEM((NBL,), jnp.float32),
            pltpu.VMEM((NBL,), jnp.float32),
            pltpu.VMEM(((NB + 1) * _L,), jnp.int32),
            pltpu.VMEM(((NB + 1) * _L,), jnp.int32),
        ],
    )
    def body(w_hbm, s_hbm, e_hbm, n_hbm, f_hbm, out_hbm,
             wblk0, wblk1, sblk0, sblk1, eblk0, eblk1, nblk0, nblk1,
             fblk0, fblk1, outblk0, outblk1, isem0, isem1, osem0, osem1,
             cdf0, cdf1, eb0, eb1, h0, h1):
        cdfs, ebinss, hists = (cdf0, cdf1), (eb0, eb1), (h0, h1)
        bufs = ((wblk0, sblk0, eblk0, nblk0, fblk0), (wblk1, sblk1, eblk1, nblk1, fblk1))
        outblks = (outblk0, outblk1)
        isems, osems = (isem0, isem1), (osem0, osem1)
        wid = lax.axis_index("s") * _NC + lax.axis_index("c")
        base = wid * RW
        lane = lax.iota(jnp.int32, _L)
        ones_i = jnp.ones((_L,), jnp.int32)
        zeros_i = jnp.zeros((_L,), jnp.int32)
        zeros_f = jnp.zeros((_L,), jnp.float32)
        def make_gpair(wblk, sblk, eblk, nblk, fblk, outblk):
          def gpair(gp, _):
            st = []
            for t in (0, 1):
                g16 = (gp * 2 + t) * _L
                rows = g16 + lane
                near = nblk[pl.ds(g16, _L)]
                far = fblk[pl.ds(g16, _L)]
                st.append(dict(
                    g16=g16, rows=rows, rowsS=rows * S, rowsO=rows * NO,
                    near=near, fmn=far - near,
                    cdf=cdfs[t], ebins=ebinss[t], hist=hists[t],
                ))

            def pa(s, accs):
                out = []
                for t in (0, 1):
                    d = st[t]
                    ww = plsc.load_gather(wblk, [d["rowsS"] + s])
                    ee = plsc.load_gather(sblk, [d["rowsS"] + s])
                    acc = accs[t] + (ww + 0.01)
                    d["cdf"][pl.ds((s + 1) * _L, _L)] = acc
                    d["ebins"][pl.ds(s * _L, _L)] = ee
                    d["hist"][pl.ds(s * _L, _L)] = zeros_i
                    out.append(acc)
                return tuple(out)

            wss = plsc.parallel_loop(0, S, 1, unroll=4, carry=(zeros_f, zeros_f))(pa)
            pads = []
            for t in (0, 1):
                d = st[t]
                d["hist"][pl.ds(S * _L, _L)] = zeros_i
                d["hist"][pl.ds((S + 1) * _L, _L)] = zeros_i
                d["cdf"][pl.ds(0, _L)] = zeros_f
                d["ebins"][pl.ds(S * _L, _L)] = eblk[pl.ds(d["g16"], _L)]
                ws = wss[t]
                pad = jnp.maximum(0.0, 1e-5 - ws)
                pads.append((pad * (1.0 / S), 1.0 / (ws + pad)))

            def pb(k, carry):
                kf = k.astype(jnp.float32)
                for t in (0, 1):
                    d = st[t]
                    pad64, inv = pads[t]
                    cum = d["cdf"][pl.ds(k * _L, _L)]
                    c = jnp.minimum(1.0, (cum + kf * pad64) * inv)
                    d["cdf"][pl.ds(k * _L, _L)] = c
                    y = c * NB - 0.5
                    tr = y.astype(jnp.int32)
                    m = jnp.where(y > tr.astype(jnp.float32), tr + 1, tr)
                    plsc.addupdate_scatter(d["hist"], [m * _L + lane], ones_i)
                    ek = d["ebins"][pl.ds(k * _L, _L)]
                    val = d["near"] + ek * d["fmn"]
                    plsc.store_scatter(outblk, [d["rowsO"] + (m + k)], val)
                return carry

            plsc.parallel_loop(0, NB, 1, unroll=4, carry=jnp.int32(0))(pb)

            def pc(j, accs):
                u = (j.astype(jnp.float32) * 2.0 + 1.0) * (1.0 / NO)
                jp1 = j + 1
                out = []
                for t in (0, 1):
                    d = st[t]
                    acc = accs[t] + d["hist"][pl.ds(j * _L, _L)]
                    below = jnp.minimum(acc - 1, S)
                    above = jnp.minimum(acc, S)
                    bidx = below * _L + lane
                    aidx = above * _L + lane
                    c0 = plsc.load_gather(d["cdf"], [bidx])
                    c1 = plsc.load_gather(d["cdf"], [aidx])
                    e0 = plsc.load_gather(d["ebins"], [bidx])
                    e1 = plsc.load_gather(d["ebins"], [aidx])
                    num = u - c0
                    den = c1 - c0
                    tt = jnp.clip(num / den, 0.0, 1.0)
                    tt = jnp.where(den == 0.0, jnp.where(num > 0.0, 1.0, 0.0), tt)
                    bval = e0 + tt * (e1 - e0)
                    val = d["near"] + bval * d["fmn"]
                    plsc.store_scatter(outblk, [d["rowsO"] + (below + jp1)], val)
                    out.append(acc)
                return tuple(out)

            plsc.parallel_loop(0, NB, 1, unroll=4, carry=(zeros_i, zeros_i))(pc)
            return 0
          return gpair

        gpairs = tuple(make_gpair(*bufs[p], outblks[p]) for p in (0, 1))

        def start_in(p, sb):
            wblk, sblk, eblk, nblk, fblk = bufs[p]

            @pl.when(sb < NSB)
            def _():
                row0 = base + sb * SB
                pltpu.async_copy(w_hbm.at[pl.ds(row0 * S, SB * S)], wblk, isems[p])
                pltpu.async_copy(s_hbm.at[pl.ds(row0 * S, SB * S)], sblk, isems[p])
                pltpu.async_copy(e_hbm.at[pl.ds(row0, SB)], eblk, isems[p])
                pltpu.async_copy(n_hbm.at[pl.ds(row0, SB)], nblk, isems[p])
                pltpu.async_copy(f_hbm.at[pl.ds(row0, SB)], fblk, isems[p])

        def wait_in(p):
            wblk, sblk, eblk, nblk, fblk = bufs[p]
            pltpu.make_async_copy(w_hbm.at[pl.ds(0, SB * S)], wblk, isems[p]).wait()
            pltpu.make_async_copy(s_hbm.at[pl.ds(0, SB * S)], sblk, isems[p]).wait()
            pltpu.make_async_copy(e_hbm.at[pl.ds(0, SB)], eblk, isems[p]).wait()
            pltpu.make_async_copy(n_hbm.at[pl.ds(0, SB)], nblk, isems[p]).wait()
            pltpu.make_async_copy(f_hbm.at[pl.ds(0, SB)], fblk, isems[p]).wait()

        def wait_out(p):
            pltpu.make_async_copy(
                w_hbm.at[pl.ds(0, SB * NO)], outblks[p], osems[p]).wait()

        start_in(0, base * 0)

        def halfstep(h, _):
            for p in (0, 1):
                sb = h * 2 + p
                start_in(1 - p, sb + 1)
                wait_in(p)

                @pl.when(h > 0)
                def _():
                    wait_out(p)

                lax.fori_loop(0, GPB // 2, gpairs[p], 0)
                row0 = base + sb * SB
                pltpu.async_copy(
                    outblks[p], out_hbm.at[pl.ds(row0 * NO, SB * NO)], osems[p])
            return 0

        lax.fori_loop(0, NSB // 2, halfstep, 0)
        wait_out(0)
        wait_out(1)

    return body(w2, s2, elast, nvec, fvec).reshape(R, NO)


def kernel(weights, spacing_starts, spacing_ends, nears, fars, num_samples=64):
    R, S = weights.shape[0], weights.shape[1]
    w2 = weights.reshape(R * S)
    s2 = spacing_starts.reshape(R * S)
    elast = spacing_ends[:, -1, 0]
    return _run(w2, s2, elast, nears.reshape(R), fars.reshape(R), S)


# speedup vs baseline: 1.1210x; 1.0555x over previous
import functools

import jax
import jax.numpy as jnp
from jax import lax
from jax.experimental import pallas as pl
from jax.experimental.pallas import tpu as pltpu
from jax.experimental.pallas import tpu_sc as plsc

_L = 16
_NC = 2
_NS = 16
_NW = _NC * _NS


@functools.partial(jax.jit, static_argnums=(5,))
def _run(w2, s2, e2, nvec, fvec, S):
    R = w2.shape[0]
    NB = S + 1
    NO = 2 * NB
    RW = R // _NW
    SB = 64
    NSB = RW // SB
    GPB = SB // _L
    NBL = NB * _L

    mesh = plsc.VectorSubcoreMesh(core_axis_name="c", subcore_axis_name="s")

    @functools.partial(
        pl.kernel,
        out_type=jax.ShapeDtypeStruct((R * NO,), jnp.float32),
        mesh=mesh,
        compiler_params=pltpu.CompilerParams(needs_layout_passes=False),
        scratch_types=[
            pltpu.VMEM((SB, S), jnp.float32),
            pltpu.VMEM((SB, S), jnp.float32),
            pltpu.VMEM((SB, S), jnp.float32),
            pltpu.VMEM((SB, S), jnp.float32),
            pltpu.VMEM((SB, S), jnp.float32),
            pltpu.VMEM((SB, S), jnp.float32),
            pltpu.VMEM((SB,), jnp.float32),
            pltpu.VMEM((SB,), jnp.float32),
            pltpu.VMEM((SB,), jnp.float32),
            pltpu.VMEM((SB,), jnp.float32),
            pltpu.VMEM((SB * NO,), jnp.float32),
            pltpu.VMEM((SB * NO,), jnp.float32),
            pltpu.SemaphoreType.DMA,
            pltpu.SemaphoreType.DMA,
            pltpu.SemaphoreType.DMA,
            pltpu.SemaphoreType.DMA,
            pltpu.VMEM((NBL,), jnp.float32),
            pltpu.VMEM((NBL,), jnp.float32),
            pltpu.VMEM((NBL,), jnp.float32),
            pltpu.VMEM((NBL,), jnp.float32),
            pltpu.VMEM(((NB + 1) * _L,), jnp.int32),
            pltpu.VMEM(((NB + 1) * _L,), jnp.int32),
        ],
    )
    def body(w_hbm, s_hbm, e_hbm, n_hbm, f_hbm, out_hbm,
             wblk0, wblk1, sblk0, sblk1, eblk0, eblk1, nblk0, nblk1,
             fblk0, fblk1, outblk0, outblk1, isem0, isem1, osem0, osem1,
             cdf0, cdf1, eb0, eb1, h0, h1):
        cdfs, ebinss, hists = (cdf0, cdf1), (eb0, eb1), (h0, h1)
        bufs = ((wblk0, sblk0, eblk0, nblk0, fblk0), (wblk1, sblk1, eblk1, nblk1, fblk1))
        outblks = (outblk0, outblk1)
        isems, osems = (isem0, isem1), (osem0, osem1)
        wid = lax.axis_index("s") * _NC + lax.axis_index("c")
        base = wid * RW
        lane = lax.iota(jnp.int32, _L)
        ones_i = jnp.ones((_L,), jnp.int32)
        zeros_i = jnp.zeros((_L,), jnp.int32)
        zeros_f = jnp.zeros((_L,), jnp.float32)
        lastcol = jnp.full((_L,), S - 1, jnp.int32)
        def make_gpair(wblk, sblk, eblk, nblk, fblk, outblk):
          def gpair(gp, _):
            st = []
            for t in (0, 1):
                g16 = (gp * 2 + t) * _L
                rows = g16 + lane
                near = nblk[pl.ds(g16, _L)]
                far = fblk[pl.ds(g16, _L)]
                st.append(dict(
                    g16=g16, rows=rows, rowsS=rows * S, rowsO=rows * NO,
                    near=near, fmn=far - near,
                    cdf=cdfs[t], ebins=ebinss[t], hist=hists[t],
                ))

            def pa(s, accs):
                out = []
                cols = lax.broadcast(s, (_L,))
                for t in (0, 1):
                    d = st[t]
                    ww = plsc.load_gather(wblk, [d["rows"], cols])
                    ee = plsc.load_gather(sblk, [d["rows"], cols])
                    acc = accs[t] + (ww + 0.01)
                    d["cdf"][pl.ds((s + 1) * _L, _L)] = acc
                    d["ebins"][pl.ds(s * _L, _L)] = ee
                    d["hist"][pl.ds(s * _L, _L)] = zeros_i
                    out.append(acc)
                return tuple(out)

            wss = plsc.parallel_loop(0, S, 1, unroll=4, carry=(zeros_f, zeros_f))(pa)
            pads = []
            for t in (0, 1):
                d = st[t]
                d["hist"][pl.ds(S * _L, _L)] = zeros_i
                d["hist"][pl.ds((S + 1) * _L, _L)] = zeros_i
                d["cdf"][pl.ds(0, _L)] = zeros_f
                e64 = plsc.load_gather(eblk, [d["rows"], lastcol])
                d["ebins"][pl.ds(S * _L, _L)] = e64
                ws = wss[t]
                pad = jnp.maximum(0.0, 1e-5 - ws)
                pads.append((pad * (1.0 / S), 1.0 / (ws + pad)))

            def pb(k, carry):
                kf = k.astype(jnp.float32)
                for t in (0, 1):
                    d = st[t]
                    pad64, inv = pads[t]
                    cum = d["cdf"][pl.ds(k * _L, _L)]
                    c = jnp.minimum(1.0, (cum + kf * pad64) * inv)
                    d["cdf"][pl.ds(k * _L, _L)] = c
                    y = c * NB - 0.5
                    tr = y.astype(jnp.int32)
                    m = jnp.where(y > tr.astype(jnp.float32), tr + 1, tr)
                    plsc.addupdate_scatter(d["hist"], [m * _L + lane], ones_i)
                    ek = d["ebins"][pl.ds(k * _L, _L)]
                    val = d["near"] + ek * d["fmn"]
                    plsc.store_scatter(outblk, [d["rowsO"] + (m + k)], val)
                return carry

            plsc.parallel_loop(0, NB, 1, unroll=4, carry=jnp.int32(0))(pb)

            def pc(j, accs):
                u = (j.astype(jnp.float32) * 2.0 + 1.0) * (1.0 / NO)
                jp1 = j + 1
                out = []
                for t in (0, 1):
                    d = st[t]
                    acc = accs[t] + d["hist"][pl.ds(j * _L, _L)]
                    below = jnp.minimum(acc - 1, S)
                    above = jnp.minimum(acc, S)
                    bidx = below * _L + lane
                    aidx = above * _L + lane
                    c0 = plsc.load_gather(d["cdf"], [bidx])
                    c1 = plsc.load_gather(d["cdf"], [aidx])
                    e0 = plsc.load_gather(d["ebins"], [bidx])
                    e1 = plsc.load_gather(d["ebins"], [aidx])
                    num = u - c0
                    den = c1 - c0
                    tt = jnp.clip(num / den, 0.0, 1.0)
                    tt = jnp.where(den == 0.0, jnp.where(num > 0.0, 1.0, 0.0), tt)
                    bval = e0 + tt * (e1 - e0)
                    val = d["near"] + bval * d["fmn"]
                    plsc.store_scatter(outblk, [d["rowsO"] + (below + jp1)], val)
                    out.append(acc)
                return tuple(out)

            plsc.parallel_loop(0, NB, 1, unroll=4, carry=(zeros_i, zeros_i))(pc)
            return 0
          return gpair

        gpairs = tuple(make_gpair(*bufs[p], outblks[p]) for p in (0, 1))

        def start_in(p, sb):
            wblk, sblk, eblk, nblk, fblk = bufs[p]

            @pl.when(sb < NSB)
            def _():
                row0 = base + sb * SB
                pltpu.async_copy(w_hbm.at[pl.ds(row0, SB)], wblk, isems[p])
                pltpu.async_copy(s_hbm.at[pl.ds(row0, SB)], sblk, isems[p])
                pltpu.async_copy(e_hbm.at[pl.ds(row0, SB)], eblk, isems[p])
                pltpu.async_copy(n_hbm.at[pl.ds(row0, SB)], nblk, isems[p])
                pltpu.async_copy(f_hbm.at[pl.ds(row0, SB)], fblk, isems[p])

        def wait_in(p):
            wblk, sblk, eblk, nblk, fblk = bufs[p]
            pltpu.make_async_copy(w_hbm.at[pl.ds(0, SB)], wblk, isems[p]).wait()
            pltpu.make_async_copy(s_hbm.at[pl.ds(0, SB)], sblk, isems[p]).wait()
            pltpu.make_async_copy(e_hbm.at[pl.ds(0, SB)], eblk, isems[p]).wait()
            pltpu.make_async_copy(n_hbm.at[pl.ds(0, SB)], nblk, isems[p]).wait()
            pltpu.make_async_copy(f_hbm.at[pl.ds(0, SB)], fblk, isems[p]).wait()

        def wait_out(p):
            pltpu.make_async_copy(
                out_hbm.at[pl.ds(0, SB * NO)], outblks[p], osems[p]).wait()

        start_in(0, base * 0)

        def halfstep(h, _):
            for p in (0, 1):
                sb = h * 2 + p
                start_in(1 - p, sb + 1)
                wait_in(p)

                @pl.when(h > 0)
                def _():
                    wait_out(p)

                lax.fori_loop(0, GPB // 2, gpairs[p], 0)
                row0 = base + sb * SB
                pltpu.async_copy(
                    outblks[p], out_hbm.at[pl.ds(row0 * NO, SB * NO)], osems[p])
            return 0

        lax.fori_loop(0, NSB // 2, halfstep, 0)
        wait_out(0)
        wait_out(1)

    return body(w2, s2, e2, nvec, fvec).reshape(R, NO)


def kernel(weights, spacing_starts, spacing_ends, nears, fars, num_samples=64):
    R, S = weights.shape[0], weights.shape[1]
    return _run(weights[:, :, 0], spacing_starts[:, :, 0], spacing_ends[:, :, 0],
                nears.reshape(R), fars.reshape(R), S)


# speedup vs baseline: 1.2667x; 1.1299x over previous
import functools

import jax
import jax.numpy as jnp
from jax import lax
from jax.experimental import pallas as pl
from jax.experimental.pallas import tpu as pltpu
from jax.experimental.pallas import tpu_sc as plsc

_L = 16
_NC = 2
_NS = 16
_NW = _NC * _NS


@functools.partial(jax.jit, static_argnums=(5,))
def _run(w2, s2, elast, nvec, fvec, S):
    R = w2.shape[0] // S
    NB = S + 1
    NO = 2 * NB
    RW = R // _NW
    SB = 128
    NSB = RW // SB
    GPB = SB // _L
    NBL = NB * _L

    mesh = plsc.VectorSubcoreMesh(core_axis_name="c", subcore_axis_name="s")

    @functools.partial(
        pl.kernel,
        out_type=jax.ShapeDtypeStruct((R, NO), jnp.float32),
        mesh=mesh,
        compiler_params=pltpu.CompilerParams(needs_layout_passes=False),
        scratch_types=[
            pltpu.VMEM((SB * S,), jnp.float32),
            pltpu.VMEM((SB * S,), jnp.float32),
            pltpu.VMEM((SB * S,), jnp.float32),
            pltpu.VMEM((SB * S,), jnp.float32),
            pltpu.VMEM((SB,), jnp.float32),
            pltpu.VMEM((SB,), jnp.float32),
            pltpu.VMEM((SB,), jnp.float32),
            pltpu.VMEM((SB,), jnp.float32),
            pltpu.VMEM((SB,), jnp.float32),
            pltpu.VMEM((SB,), jnp.float32),
            pltpu.VMEM((SB, NO), jnp.float32),
            pltpu.VMEM((SB, NO), jnp.float32),
            pltpu.SemaphoreType.DMA,
            pltpu.SemaphoreType.DMA,
            pltpu.SemaphoreType.DMA,
            pltpu.SemaphoreType.DMA,
            pltpu.VMEM((NBL,), jnp.float32),
            pltpu.VMEM((NBL,), jnp.float32),
            pltpu.VMEM((NBL,), jnp.float32),
            pltpu.VMEM((NBL,), jnp.float32),
            pltpu.VMEM(((NB + 1) * _L,), jnp.int32),
            pltpu.VMEM(((NB + 1) * _L,), jnp.int32),
        ],
    )
    def body(w_hbm, s_hbm, e_hbm, n_hbm, f_hbm, out_hbm,
             wblk0, wblk1, sblk0, sblk1, eblk0, eblk1, nblk0, nblk1,
             fblk0, fblk1, outblk0, outblk1, isem0, isem1, osem0, osem1,
             cdf0, cdf1, eb0, eb1, h0, h1):
        cdfs, ebinss, hists = (cdf0, cdf1), (eb0, eb1), (h0, h1)
        bufs = ((wblk0, sblk0, eblk0, nblk0, fblk0), (wblk1, sblk1, eblk1, nblk1, fblk1))
        outblks = (outblk0, outblk1)
        isems, osems = (isem0, isem1), (osem0, osem1)
        wid = lax.axis_index("s") * _NC + lax.axis_index("c")
        base = wid * RW
        lane = lax.iota(jnp.int32, _L)
        ones_i = jnp.ones((_L,), jnp.int32)
        zeros_i = jnp.zeros((_L,), jnp.int32)
        zeros_f = jnp.zeros((_L,), jnp.float32)
        def make_gpair(wblk, sblk, eblk, nblk, fblk, outblk):
          def gpair(gp, _):
            st = []
            for t in (0, 1):
                g16 = (gp * 2 + t) * _L
                rows = g16 + lane
                near = nblk[pl.ds(g16, _L)]
                far = fblk[pl.ds(g16, _L)]
                st.append(dict(
                    g16=g16, rows=rows, rowsS=rows * S,
                    near=near, fmn=far - near,
                    cdf=cdfs[t], ebins=ebinss[t], hist=hists[t],
                ))

            def pa(s, accs):
                out = []
                for t in (0, 1):
                    d = st[t]
                    ww = plsc.load_gather(wblk, [d["rowsS"] + s])
                    ee = plsc.load_gather(sblk, [d["rowsS"] + s])
                    acc = accs[t] + (ww + 0.01)
                    d["cdf"][pl.ds((s + 1) * _L, _L)] = acc
                    d["ebins"][pl.ds(s * _L, _L)] = ee
                    d["hist"][pl.ds(s * _L, _L)] = zeros_i
                    out.append(acc)
                return tuple(out)

            wss = plsc.parallel_loop(0, S, 1, unroll=4, carry=(zeros_f, zeros_f))(pa)
            pads = []
            for t in (0, 1):
                d = st[t]
                d["hist"][pl.ds(S * _L, _L)] = zeros_i
                d["hist"][pl.ds((S + 1) * _L, _L)] = zeros_i
                d["cdf"][pl.ds(0, _L)] = zeros_f
                d["ebins"][pl.ds(S * _L, _L)] = eblk[pl.ds(d["g16"], _L)]
                ws = wss[t]
                pad = jnp.maximum(0.0, 1e-5 - ws)
                pads.append((pad * (1.0 / S), 1.0 / (ws + pad)))

            def pb(k, carry):
                kf = k.astype(jnp.float32)
                for t in (0, 1):
                    d = st[t]
                    pad64, inv = pads[t]
                    cum = d["cdf"][pl.ds(k * _L, _L)]
                    c = jnp.minimum(1.0, (cum + kf * pad64) * inv)
                    d["cdf"][pl.ds(k * _L, _L)] = c
                    y = c * NB - 0.5
                    tr = y.astype(jnp.int32)
                    m = jnp.where(y > tr.astype(jnp.float32), tr + 1, tr)
                    plsc.addupdate_scatter(d["hist"], [m * _L + lane], ones_i)
                    ek = d["ebins"][pl.ds(k * _L, _L)]
                    val = d["near"] + ek * d["fmn"]
                    plsc.store_scatter(outblk, [d["rows"], m + k], val)
                return carry

            plsc.parallel_loop(0, NB, 1, unroll=4, carry=jnp.int32(0))(pb)

            def pc(j, accs):
                u = (j.astype(jnp.float32) * 2.0 + 1.0) * (1.0 / NO)
                jp1 = j + 1
                out = []
                for t in (0, 1):
                    d = st[t]
                    acc = accs[t] + d["hist"][pl.ds(j * _L, _L)]
                    below = jnp.minimum(acc - 1, S)
                    above = jnp.minimum(acc, S)
                    bidx = below * _L + lane
                    aidx = above * _L + lane
                    c0 = plsc.load_gather(d["cdf"], [bidx])
                    c1 = plsc.load_gather(d["cdf"], [aidx])
                    e0 = plsc.load_gather(d["ebins"], [bidx])
                    e1 = plsc.load_gather(d["ebins"], [aidx])
                    num = u - c0
                    den = c1 - c0
                    tt = jnp.clip(num / den, 0.0, 1.0)
                    tt = jnp.where(den == 0.0, jnp.where(num > 0.0, 1.0, 0.0), tt)
                    bval = e0 + tt * (e1 - e0)
                    val = d["near"] + bval * d["fmn"]
                    plsc.store_scatter(outblk, [d["rows"], below + jp1], val)
                    out.append(acc)
                return tuple(out)

            plsc.parallel_loop(0, NB, 1, unroll=4, carry=(zeros_i, zeros_i))(pc)
            return 0
          return gpair

        gpairs = tuple(make_gpair(*bufs[p], outblks[p]) for p in (0, 1))

        def start_in(p, sb):
            wblk, sblk, eblk, nblk, fblk = bufs[p]

            @pl.when(sb < NSB)
            def _():
                row0 = base + sb * SB
                pltpu.async_copy(w_hbm.at[pl.ds(row0 * S, SB * S)], wblk, isems[p])
                pltpu.async_copy(s_hbm.at[pl.ds(row0 * S, SB * S)], sblk, isems[p])
                pltpu.async_copy(e_hbm.at[pl.ds(row0, SB)], eblk, isems[p])
                pltpu.async_copy(n_hbm.at[pl.ds(row0, SB)], nblk, isems[p])
                pltpu.async_copy(f_hbm.at[pl.ds(row0, SB)], fblk, isems[p])

        def wait_in(p):
            wblk, sblk, eblk, nblk, fblk = bufs[p]
            pltpu.make_async_copy(w_hbm.at[pl.ds(0, SB * S)], wblk, isems[p]).wait()
            pltpu.make_async_copy(s_hbm.at[pl.ds(0, SB * S)], sblk, isems[p]).wait()
            pltpu.make_async_copy(e_hbm.at[pl.ds(0, SB)], eblk, isems[p]).wait()
            pltpu.make_async_copy(n_hbm.at[pl.ds(0, SB)], nblk, isems[p]).wait()
            pltpu.make_async_copy(f_hbm.at[pl.ds(0, SB)], fblk, isems[p]).wait()

        def wait_out(p):
            pltpu.make_async_copy(
                out_hbm.at[pl.ds(0, SB)], outblks[p], osems[p]).wait()

        start_in(0, base * 0)

        def halfstep(h, _):
            for p in (0, 1):
                sb = h * 2 + p
                start_in(1 - p, sb + 1)
                wait_in(p)

                @pl.when(h > 0)
                def _():
                    wait_out(p)

                lax.fori_loop(0, GPB // 2, gpairs[p], 0)
                row0 = base + sb * SB
                pltpu.async_copy(outblks[p], out_hbm.at[pl.ds(row0, SB)], osems[p])
            return 0

        lax.fori_loop(0, NSB // 2, halfstep, 0)
        wait_out(0)
        wait_out(1)

    return body(w2, s2, elast, nvec, fvec)


def kernel(weights, spacing_starts, spacing_ends, nears, fars, num_samples=64):
    R, S = weights.shape[0], weights.shape[1]
    w2 = weights.reshape(R * S)
    s2 = spacing_starts.reshape(R * S)
    elast = spacing_ends[:, -1, 0]
    return _run(w2, s2, elast, nears.reshape(R), fars.reshape(R), S)


# speedup vs baseline: 2.3221x; 1.8332x over previous
import functools

import jax
import jax.numpy as jnp
from jax import lax
from jax.experimental import pallas as pl
from jax.experimental.pallas import tpu as pltpu
from jax.experimental.pallas import tpu_sc as plsc

_L = 16
_NC = 2
_NS = 16
_NW = _NC * _NS


@functools.partial(jax.jit, static_argnums=(5,))
def _run(wf, sf, ef, nf, ff, S):
    R = wf.shape[0] // S
    NB = S + 1
    NO = 2 * NB
    RW = R // _NW
    SB = 128
    NSB = RW // SB
    GPB = SB // _L
    NBL = NB * _L

    mesh = plsc.VectorSubcoreMesh(core_axis_name="c", subcore_axis_name="s")

    @functools.partial(
        pl.kernel,
        out_type=jax.ShapeDtypeStruct((NO, R), jnp.float32),
        mesh=mesh,
        compiler_params=pltpu.CompilerParams(needs_layout_passes=False),
        scratch_types=[
            pltpu.VMEM((S * SB,), jnp.float32),
            pltpu.VMEM((S * SB,), jnp.float32),
            pltpu.VMEM(((S + 1) * SB,), jnp.float32),
            pltpu.VMEM(((S + 1) * SB,), jnp.float32),
            pltpu.VMEM((SB,), jnp.float32),
            pltpu.VMEM((SB,), jnp.float32),
            pltpu.VMEM((SB,), jnp.float32),
            pltpu.VMEM((SB,), jnp.float32),
            pltpu.VMEM((NO, SB), jnp.float32),
            pltpu.VMEM((NO, SB), jnp.float32),
            pltpu.SemaphoreType.DMA,
            pltpu.SemaphoreType.DMA,
            pltpu.SemaphoreType.DMA,
            pltpu.SemaphoreType.DMA,
            pltpu.VMEM((NBL,), jnp.float32),
            pltpu.VMEM((NBL,), jnp.float32),
            pltpu.VMEM(((NB + 1) * _L,), jnp.int32),
            pltpu.VMEM(((NB + 1) * _L,), jnp.int32),
        ],
    )
    def body(w_hbm, s_hbm, e_hbm, n_hbm, f_hbm, out_hbm,
             wblk0, wblk1, sblk0, sblk1, nblk0, nblk1, fblk0, fblk1,
             outblk0, outblk1, isem0, isem1, osem0, osem1,
             cdf0, cdf1, h0, h1):
        cdfs, hists = (cdf0, cdf1), (h0, h1)
        bufs = ((wblk0, sblk0, nblk0, fblk0), (wblk1, sblk1, nblk1, fblk1))
        outblks = (outblk0, outblk1)
        isems, osems = (isem0, isem1), (osem0, osem1)
        wid = lax.axis_index("s") * _NC + lax.axis_index("c")
        base = wid * RW
        lane = lax.iota(jnp.int32, _L)
        ones_i = jnp.ones((_L,), jnp.int32)
        zeros_i = jnp.zeros((_L,), jnp.int32)
        zeros_f = jnp.zeros((_L,), jnp.float32)

        def make_gpair(wblk, sblk, nblk, fblk, outblk):
          def gpair(gp, _):
            st = []
            for t in (0, 1):
                g16 = (gp * 2 + t) * _L
                rows = g16 + lane
                near = nblk[pl.ds(g16, _L)]
                far = fblk[pl.ds(g16, _L)]
                st.append(dict(
                    g16=g16, rows=rows, rowsm=rows - SB,
                    near=near, fmn=far - near, cdf=cdfs[t], hist=hists[t],
                ))

            def pa(s, accs):
                out = []
                sSB = s * SB
                for t in (0, 1):
                    d = st[t]
                    ww = wblk[pl.ds(sSB + d["g16"], _L)]
                    acc = accs[t] + (ww + 0.01)
                    d["cdf"][pl.ds((s + 1) * _L, _L)] = acc
                    d["hist"][pl.ds(s * _L, _L)] = zeros_i
                    out.append(acc)
                return tuple(out)

            wss = plsc.parallel_loop(0, S, 1, unroll=4, carry=(zeros_f, zeros_f))(pa)
            pads = []
            for t in (0, 1):
                d = st[t]
                d["hist"][pl.ds(S * _L, _L)] = zeros_i
                d["hist"][pl.ds((S + 1) * _L, _L)] = zeros_i
                d["hist"][pl.ds(0, _L)] = ones_i
                d["cdf"][pl.ds(0, _L)] = zeros_f
                ws = wss[t]
                pad = jnp.maximum(0.0, 1e-5 - ws)
                pads.append((pad * (1.0 / S), 1.0 / (ws + pad)))

            def pb(k, carry):
                kf = k.astype(jnp.float32)
                kSB = k * SB
                for t in (0, 1):
                    d = st[t]
                    pad64, inv = pads[t]
                    cum = d["cdf"][pl.ds(k * _L, _L)]
                    c = jnp.minimum(1.0, (cum + kf * pad64) * inv)
                    d["cdf"][pl.ds(k * _L, _L)] = c
                    y = c * NB - 0.5
                    tr = y.astype(jnp.int32)
                    m = jnp.where(y > tr.astype(jnp.float32), tr + 1, tr)
                    plsc.addupdate_scatter(d["hist"], [m * _L + lane], ones_i)
                    ek = sblk[pl.ds(kSB + d["g16"], _L)]
                    val = d["near"] + ek * d["fmn"]
                    plsc.store_scatter(outblk, [m + k, d["rows"]], val)
                return carry

            plsc.parallel_loop(0, NB, 1, unroll=4, carry=jnp.int32(0))(pb)

            def pc(j, accs):
                u = (j.astype(jnp.float32) * 2.0 + 1.0) * (1.0 / NO)
                out = []
                for t in (0, 1):
                    d = st[t]
                    acc = accs[t] + d["hist"][pl.ds(j * _L, _L)]
                    aidx = acc * _L + lane
                    bidx = aidx - _L
                    c0 = plsc.load_gather(d["cdf"], [bidx])
                    c1 = plsc.load_gather(d["cdf"], [aidx])
                    saidx = acc * SB + d["rowsm"] + SB
                    e0 = plsc.load_gather(sblk, [saidx - SB])
                    e1 = plsc.load_gather(sblk, [saidx])
                    num = u - c0
                    den = c1 - c0
                    tt = jnp.where(num <= 0.0, 0.0,
                                   jnp.where(num >= den, 1.0, num / den))
                    bval = e0 + tt * (e1 - e0)
                    val = d["near"] + bval * d["fmn"]
                    plsc.store_scatter(outblk, [acc + j, d["rows"]], val)
                    out.append(acc)
                return tuple(out)

            plsc.parallel_loop(0, NB, 1, unroll=4, carry=(zeros_i, zeros_i))(pc)
            return 0
          return gpair

        gpairs = tuple(make_gpair(*bufs[p], outblks[p]) for p in (0, 1))

        def start_in(p, sb):
            wblk, sblk, nblk, fblk = bufs[p]

            @pl.when(sb < NSB)
            def _():
                row0 = base + sb * SB

                def issue(s, _):
                    pltpu.async_copy(
                        w_hbm.at[pl.ds(s * R + row0, SB)],
                        wblk.at[pl.ds(s * SB, SB)], isems[p])
                    pltpu.async_copy(
                        s_hbm.at[pl.ds(s * R + row0, SB)],
                        sblk.at[pl.ds(s * SB, SB)], isems[p])
                    return 0

                lax.fori_loop(0, S, issue, 0)
                pltpu.async_copy(
                    e_hbm.at[pl.ds(row0, SB)], sblk.at[pl.ds(S * SB, SB)],
                    isems[p])
                pltpu.async_copy(n_hbm.at[pl.ds(row0, SB)], nblk, isems[p])
                pltpu.async_copy(f_hbm.at[pl.ds(row0, SB)], fblk, isems[p])

        def wait_in(p):
            wblk, sblk, nblk, fblk = bufs[p]

            def drain(s, _):
                pltpu.make_async_copy(
                    w_hbm.at[pl.ds(0, SB)], wblk.at[pl.ds(s * SB, SB)],
                    isems[p]).wait()
                pltpu.make_async_copy(
                    w_hbm.at[pl.ds(0, SB)], sblk.at[pl.ds(s * SB, SB)],
                    isems[p]).wait()
                return 0

            lax.fori_loop(0, S, drain, 0)
            pltpu.make_async_copy(
                w_hbm.at[pl.ds(0, SB)], sblk.at[pl.ds(S * SB, SB)],
                isems[p]).wait()
            pltpu.make_async_copy(w_hbm.at[pl.ds(0, SB)], nblk, isems[p]).wait()
            pltpu.make_async_copy(w_hbm.at[pl.ds(0, SB)], fblk, isems[p]).wait()

        def wait_out(p):
            pltpu.make_async_copy(
                out_hbm.at[:, pl.ds(0, SB)], outblks[p], osems[p]).wait()

        start_in(0, 0)

        def halfstep(h, _):
            for p in (0, 1):
                sb = h * 2 + p
                start_in(1 - p, sb + 1)
                wait_in(p)

                @pl.when(h > 0)
                def _():
                    wait_out(p)

                lax.fori_loop(0, GPB // 2, gpairs[p], 0)
                row0 = base + sb * SB
                pltpu.async_copy(
                    outblks[p], out_hbm.at[:, pl.ds(row0, SB)], osems[p])
            return 0

        lax.fori_loop(0, NSB // 2, halfstep, 0)
        wait_out(0)
        wait_out(1)

    return body(wf, sf, ef, nf, ff).T


def kernel(weights, spacing_starts, spacing_ends, nears, fars, num_samples=64):
    R, S = weights.shape[0], weights.shape[1]
    wf = weights[:, :, 0].T.reshape(S * R)
    sf = spacing_starts[:, :, 0].T.reshape(S * R)
    ef = spacing_ends[:, -1, 0]
    return _run(wf, sf, ef, nears.reshape(R), fars.reshape(R), S)


# speedup vs baseline: 2.3222x; 1.0000x over previous
import functools

import jax
import jax.numpy as jnp
from jax import lax
from jax.experimental import pallas as pl
from jax.experimental.pallas import tpu as pltpu
from jax.experimental.pallas import tpu_sc as plsc

_L = 16
_NC = 2
_NS = 16
_NW = _NC * _NS


@functools.partial(jax.jit, static_argnums=(5,))
def _run(wf, sf, ef, nf, ff, S):
    R = wf.shape[0] // S
    NB = S + 1
    NO = 2 * NB
    RW = R // _NW
    SB = 128
    NSB = RW // SB
    GPB = SB // _L
    NBL = NB * _L

    mesh = plsc.VectorSubcoreMesh(core_axis_name="c", subcore_axis_name="s")

    @functools.partial(
        pl.kernel,
        out_type=jax.ShapeDtypeStruct((NO, R), jnp.float32),
        mesh=mesh,
        compiler_params=pltpu.CompilerParams(needs_layout_passes=False),
        scratch_types=[
            pltpu.VMEM((S * SB,), jnp.float32),
            pltpu.VMEM((S * SB,), jnp.float32),
            pltpu.VMEM(((S + 1) * SB,), jnp.float32),
            pltpu.VMEM(((S + 1) * SB,), jnp.float32),
            pltpu.VMEM((SB,), jnp.float32),
            pltpu.VMEM((SB,), jnp.float32),
            pltpu.VMEM((SB,), jnp.float32),
            pltpu.VMEM((SB,), jnp.float32),
            pltpu.VMEM((NO, SB), jnp.float32),
            pltpu.VMEM((NO, SB), jnp.float32),
            pltpu.SemaphoreType.DMA,
            pltpu.SemaphoreType.DMA,
            pltpu.SemaphoreType.DMA,
            pltpu.SemaphoreType.DMA,
            pltpu.VMEM((NBL,), jnp.float32),
            pltpu.VMEM((NBL,), jnp.float32),
            pltpu.VMEM(((NB + 1) * _L,), jnp.int32),
            pltpu.VMEM(((NB + 1) * _L,), jnp.int32),
        ],
    )
    def body(w_hbm, s_hbm, e_hbm, n_hbm, f_hbm, out_hbm,
             wblk0, wblk1, sblk0, sblk1, nblk0, nblk1, fblk0, fblk1,
             outblk0, outblk1, isem0, isem1, osem0, osem1,
             cdf0, cdf1, h0, h1):
        cdfs, hists = (cdf0, cdf1), (h0, h1)
        bufs = ((wblk0, sblk0, nblk0, fblk0), (wblk1, sblk1, nblk1, fblk1))
        outblks = (outblk0, outblk1)
        isems, osems = (isem0, isem1), (osem0, osem1)
        wid = lax.axis_index("s") * _NC + lax.axis_index("c")
        base = wid * RW
        lane = lax.iota(jnp.int32, _L)
        ones_i = jnp.ones((_L,), jnp.int32)
        zeros_i = jnp.zeros((_L,), jnp.int32)
        zeros_f = jnp.zeros((_L,), jnp.float32)

        def make_gpair(wblk, sblk, nblk, fblk, outblk):
          def gpair(gp, _):
            st = []
            for t in (0, 1):
                g16 = (gp * 2 + t) * _L
                rows = g16 + lane
                near = nblk[pl.ds(g16, _L)]
                far = fblk[pl.ds(g16, _L)]
                st.append(dict(
                    g16=g16, rows=rows, rowsm=rows - SB,
                    near=near, fmn=far - near, cdf=cdfs[t], hist=hists[t],
                ))

            def pa(s, accs):
                out = []
                sSB = s * SB
                for t in (0, 1):
                    d = st[t]
                    ww = wblk[pl.ds(sSB + d["g16"], _L)]
                    acc = accs[t] + (ww + 0.01)
                    d["cdf"][pl.ds((s + 1) * _L, _L)] = acc
                    d["hist"][pl.ds(s * _L, _L)] = zeros_i
                    out.append(acc)
                return tuple(out)

            wss = plsc.parallel_loop(0, S, 1, unroll=4, carry=(zeros_f, zeros_f))(pa)
            pads = []
            for t in (0, 1):
                d = st[t]
                d["hist"][pl.ds(S * _L, _L)] = zeros_i
                d["hist"][pl.ds((S + 1) * _L, _L)] = zeros_i
                d["cdf"][pl.ds(0, _L)] = zeros_f
                ws = wss[t]
                pad = jnp.maximum(0.0, 1e-5 - ws)
                pads.append((pad * (1.0 / S), 1.0 / (ws + pad)))

            def pb(k, carry):
                kf = k.astype(jnp.float32)
                kSB = k * SB
                for t in (0, 1):
                    d = st[t]
                    pad64, inv = pads[t]
                    cum = d["cdf"][pl.ds(k * _L, _L)]
                    c = jnp.minimum(1.0, (cum + kf * pad64) * inv)
                    d["cdf"][pl.ds(k * _L, _L)] = c
                    y = c * NB - 0.5
                    tr = y.astype(jnp.int32)
                    m = jnp.where(y > tr.astype(jnp.float32), tr + 1, tr)
                    plsc.addupdate_scatter(d["hist"], [m * _L + lane], ones_i)
                    ek = sblk[pl.ds(kSB + d["g16"], _L)]
                    val = d["near"] + ek * d["fmn"]
                    plsc.store_scatter(outblk, [m + k, d["rows"]], val)
                return carry

            plsc.parallel_loop(0, NB, 1, unroll=4, carry=jnp.int32(0))(pb)

            def pc(j, accs):
                u = (j.astype(jnp.float32) * 2.0 + 1.0) * (1.0 / NO)
                out = []
                for t in (0, 1):
                    d = st[t]
                    acc = accs[t] + d["hist"][pl.ds(j * _L, _L)]
                    aidx = acc * _L + lane
                    bidx = aidx - _L
                    c0 = plsc.load_gather(d["cdf"], [bidx])
                    c1 = plsc.load_gather(d["cdf"], [aidx])
                    saidx = acc * SB + d["rowsm"] + SB
                    e0 = plsc.load_gather(sblk, [saidx - SB])
                    e1 = plsc.load_gather(sblk, [saidx])
                    num = u - c0
                    den = c1 - c0
                    tt = jnp.where(num <= 0.0, 0.0,
                                   jnp.where(num >= den, 1.0, num / den))
                    bval = e0 + tt * (e1 - e0)
                    val = d["near"] + bval * d["fmn"]
                    plsc.store_scatter(outblk, [acc + j, d["rows"]], val)
                    out.append(acc)
                return tuple(out)

            plsc.parallel_loop(0, NB, 1, unroll=4, carry=(zeros_i, zeros_i))(pc)
            return 0
          return gpair

        gpairs = tuple(make_gpair(*bufs[p], outblks[p]) for p in (0, 1))

        def start_in(p, sb):
            wblk, sblk, nblk, fblk = bufs[p]

            @pl.when(sb < NSB)
            def _():
                row0 = base + sb * SB

                def issue(s, _):
                    pltpu.async_copy(
                        w_hbm.at[pl.ds(s * R + row0, SB)],
                        wblk.at[pl.ds(s * SB, SB)], isems[p])
                    pltpu.async_copy(
                        s_hbm.at[pl.ds(s * R + row0, SB)],
                        sblk.at[pl.ds(s * SB, SB)], isems[p])
                    return 0

                lax.fori_loop(0, S, issue, 0)
                pltpu.async_copy(
                    e_hbm.at[pl.ds(row0, SB)], sblk.at[pl.ds(S * SB, SB)],
                    isems[p])
                pltpu.async_copy(n_hbm.at[pl.ds(row0, SB)], nblk, isems[p])
                pltpu.async_copy(f_hbm.at[pl.ds(row0, SB)], fblk, isems[p])

        def wait_in(p):
            wblk, sblk, nblk, fblk = bufs[p]

            def drain(s, _):
                pltpu.make_async_copy(
                    w_hbm.at[pl.ds(0, SB)], wblk.at[pl.ds(s * SB, SB)],
                    isems[p]).wait()
                pltpu.make_async_copy(
                    w_hbm.at[pl.ds(0, SB)], sblk.at[pl.ds(s * SB, SB)],
                    isems[p]).wait()
                return 0

            lax.fori_loop(0, S, drain, 0)
            pltpu.make_async_copy(
                w_hbm.at[pl.ds(0, SB)], sblk.at[pl.ds(S * SB, SB)],
                isems[p]).wait()
            pltpu.make_async_copy(w_hbm.at[pl.ds(0, SB)], nblk, isems[p]).wait()
            pltpu.make_async_copy(w_hbm.at[pl.ds(0, SB)], fblk, isems[p]).wait()

        def wait_out(p):
            pltpu.make_async_copy(
                out_hbm.at[:, pl.ds(0, SB)], outblks[p], osems[p]).wait()

        start_in(0, 0)

        def halfstep(h, _):
            for p in (0, 1):
                sb = h * 2 + p
                start_in(1 - p, sb + 1)
                wait_in(p)

                @pl.when(h > 0)
                def _():
                    wait_out(p)

                lax.fori_loop(0, GPB // 2, gpairs[p], 0)
                row0 = base + sb * SB
                pltpu.async_copy(
                    outblks[p], out_hbm.at[:, pl.ds(row0, SB)], osems[p])
            return 0

        lax.fori_loop(0, NSB // 2, halfstep, 0)
        wait_out(0)
        wait_out(1)

    return body(wf, sf, ef, nf, ff).T


def kernel(weights, spacing_starts, spacing_ends, nears, fars, num_samples=64):
    R, S = weights.shape[0], weights.shape[1]
    wf = weights[:, :, 0].T.reshape(S * R)
    sf = spacing_starts[:, :, 0].T.reshape(S * R)
    ef = spacing_ends[:, -1, 0]
    return _run(wf, sf, ef, nears.reshape(R), fars.reshape(R), S)


# speedup vs baseline: 3.1575x; 1.3597x over previous
import functools

import jax
import jax.numpy as jnp
from jax import lax
from jax.experimental import pallas as pl
from jax.experimental.pallas import tpu as pltpu
from jax.experimental.pallas import tpu_sc as plsc

_L = 16
_NC = 2
_NS = 16
_NW = _NC * _NS


@functools.partial(jax.jit, static_argnums=(5,))
def _run(wf, sf, ef, nf, ff, S):
    R = wf.shape[1]
    NB = S + 1
    NO = 2 * NB
    RW = R // _NW
    SB = 128
    NSB = RW // SB
    GPB = SB // _L
    NBL = NB * _L

    mesh = plsc.VectorSubcoreMesh(core_axis_name="c", subcore_axis_name="s")

    @functools.partial(
        pl.kernel,
        out_type=jax.ShapeDtypeStruct((NO, R), jnp.float32),
        mesh=mesh,
        compiler_params=pltpu.CompilerParams(needs_layout_passes=False),
        scratch_types=[
            pltpu.VMEM((S, SB), jnp.float32),
            pltpu.VMEM((S, SB), jnp.float32),
            pltpu.VMEM((S + 1, SB), jnp.float32),
            pltpu.VMEM((S + 1, SB), jnp.float32),
            pltpu.VMEM((SB,), jnp.float32),
            pltpu.VMEM((SB,), jnp.float32),
            pltpu.VMEM((SB,), jnp.float32),
            pltpu.VMEM((SB,), jnp.float32),
            pltpu.VMEM((NO, SB), jnp.float32),
            pltpu.VMEM((NO, SB), jnp.float32),
            pltpu.SemaphoreType.DMA,
            pltpu.SemaphoreType.DMA,
            pltpu.SemaphoreType.DMA,
            pltpu.SemaphoreType.DMA,
            pltpu.VMEM((NBL,), jnp.float32),
            pltpu.VMEM((NBL,), jnp.float32),
            pltpu.VMEM(((NB + 1) * _L,), jnp.int32),
            pltpu.VMEM(((NB + 1) * _L,), jnp.int32),
        ],
    )
    def body(w_hbm, s_hbm, e_hbm, n_hbm, f_hbm, out_hbm,
             wblk0, wblk1, sblk0, sblk1, nblk0, nblk1, fblk0, fblk1,
             outblk0, outblk1, isem0, isem1, osem0, osem1,
             cdf0, cdf1, h0, h1):
        cdfs, hists = (cdf0, cdf1), (h0, h1)
        bufs = ((wblk0, sblk0, nblk0, fblk0), (wblk1, sblk1, nblk1, fblk1))
        outblks = (outblk0, outblk1)
        isems, osems = (isem0, isem1), (osem0, osem1)
        wid = lax.axis_index("s") * _NC + lax.axis_index("c")
        base = wid * RW
        lane = lax.iota(jnp.int32, _L)
        ones_i = jnp.ones((_L,), jnp.int32)
        zeros_i = jnp.zeros((_L,), jnp.int32)
        zeros_f = jnp.zeros((_L,), jnp.float32)

        def make_gpair(wblk, sblk, nblk, fblk, outblk):
          def gpair(gp, _):
            st = []
            for t in (0, 1):
                g16 = (gp * 2 + t) * _L
                rows = g16 + lane
                near = nblk[pl.ds(g16, _L)]
                far = fblk[pl.ds(g16, _L)]
                st.append(dict(
                    g16=g16, rows=rows,
                    near=near, fmn=far - near, cdf=cdfs[t], hist=hists[t],
                ))

            def pa(s, accs):
                out = []
                for t in (0, 1):
                    d = st[t]
                    ww = wblk[s, pl.ds(d["g16"], _L)]
                    acc = accs[t] + (ww + 0.01)
                    d["cdf"][pl.ds((s + 1) * _L, _L)] = acc
                    d["hist"][pl.ds(s * _L, _L)] = zeros_i
                    out.append(acc)
                return tuple(out)

            wss = plsc.parallel_loop(0, S, 1, unroll=4, carry=(zeros_f, zeros_f))(pa)
            invs = []
            for t in (0, 1):
                d = st[t]
                d["hist"][pl.ds(S * _L, _L)] = zeros_i
                d["hist"][pl.ds((S + 1) * _L, _L)] = zeros_i
                d["cdf"][pl.ds(0, _L)] = zeros_f
                invs.append(1.0 / wss[t])

            def pb(k, carry):
                for t in (0, 1):
                    d = st[t]
                    c = d["cdf"][pl.ds(k * _L, _L)] * invs[t]
                    d["cdf"][pl.ds(k * _L, _L)] = c
                    y = c * NB - 0.5
                    tr = y.astype(jnp.int32)
                    m = jnp.where(y > tr.astype(jnp.float32), tr + 1, tr)
                    plsc.addupdate_scatter(d["hist"], [m * _L + lane], ones_i)
                    ek = sblk[k, pl.ds(d["g16"], _L)]
                    val = d["near"] + ek * d["fmn"]
                    plsc.store_scatter(outblk, [m + k, d["rows"]], val)
                return carry

            plsc.parallel_loop(0, NB, 1, unroll=4, carry=jnp.int32(0))(pb)

            def pc(j, accs):
                u = (j.astype(jnp.float32) * 2.0 + 1.0) * (1.0 / NO)
                out = []
                for t in (0, 1):
                    d = st[t]
                    acc = accs[t] + d["hist"][pl.ds(j * _L, _L)]
                    aidx = acc * _L + lane
                    bidx = aidx - _L
                    c0 = plsc.load_gather(d["cdf"], [bidx])
                    c1 = plsc.load_gather(d["cdf"], [aidx])
                    e0 = plsc.load_gather(sblk, [acc - 1, d["rows"]])
                    e1 = plsc.load_gather(sblk, [acc, d["rows"]])
                    num = u - c0
                    den = c1 - c0
                    tt = jnp.where(num <= 0.0, 0.0,
                                   jnp.where(num >= den, 1.0, num / den))
                    bval = e0 + tt * (e1 - e0)
                    val = d["near"] + bval * d["fmn"]
                    plsc.store_scatter(outblk, [acc + j, d["rows"]], val)
                    out.append(acc)
                return tuple(out)

            plsc.parallel_loop(0, NB, 1, unroll=4, carry=(zeros_i, zeros_i))(pc)
            return 0
          return gpair

        gpairs = tuple(make_gpair(*bufs[p], outblks[p]) for p in (0, 1))

        def start_in(p, sb):
            wblk, sblk, nblk, fblk = bufs[p]

            @pl.when(sb < NSB)
            def _():
                row0 = base + sb * SB
                pltpu.async_copy(w_hbm.at[:, pl.ds(row0, SB)], wblk, isems[p])
                pltpu.async_copy(
                    s_hbm.at[:, pl.ds(row0, SB)], sblk.at[pl.ds(0, S)], isems[p])
                pltpu.async_copy(
                    e_hbm.at[pl.ds(row0, SB)], sblk.at[S], isems[p])
                pltpu.async_copy(n_hbm.at[pl.ds(row0, SB)], nblk, isems[p])
                pltpu.async_copy(f_hbm.at[pl.ds(row0, SB)], fblk, isems[p])

        def wait_in(p):
            wblk, sblk, nblk, fblk = bufs[p]

            pltpu.make_async_copy(
                w_hbm.at[:, pl.ds(0, SB)], wblk, isems[p]).wait()
            pltpu.make_async_copy(
                s_hbm.at[:, pl.ds(0, SB)], sblk.at[pl.ds(0, S)], isems[p]).wait()
            pltpu.make_async_copy(
                e_hbm.at[pl.ds(0, SB)], sblk.at[S], isems[p]).wait()
            pltpu.make_async_copy(e_hbm.at[pl.ds(0, SB)], nblk, isems[p]).wait()
            pltpu.make_async_copy(e_hbm.at[pl.ds(0, SB)], fblk, isems[p]).wait()

        def wait_out(p):
            pltpu.make_async_copy(
                out_hbm.at[:, pl.ds(0, SB)], outblks[p], osems[p]).wait()

        start_in(0, 0)

        def halfstep(h, _):
            for p in (0, 1):
                sb = h * 2 + p
                start_in(1 - p, sb + 1)
                wait_in(p)

                @pl.when(h > 0)
                def _():
                    wait_out(p)

                lax.fori_loop(0, GPB // 2, gpairs[p], 0)
                row0 = base + sb * SB
                pltpu.async_copy(
                    outblks[p], out_hbm.at[:, pl.ds(row0, SB)], osems[p])
            return 0

        lax.fori_loop(0, NSB // 2, halfstep, 0)
        wait_out(0)
        wait_out(1)

    return body(wf, sf, ef, nf, ff).T


def kernel(weights, spacing_starts, spacing_ends, nears, fars, num_samples=64):
    R, S = weights.shape[0], weights.shape[1]
    wf = weights[:, :, 0].T
    sf = spacing_starts[:, :, 0].T
    ef = spacing_ends[:, -1, 0]
    return _run(wf, sf, ef, nears.reshape(R), fars.reshape(R), S)
